# baseline (speedup 1.0000x reference)
import functools

import jax
import jax.numpy as jnp
from jax import lax
from jax.experimental import pallas as pl
from jax.experimental.pallas import tpu as pltpu
from jax.experimental.pallas import tpu_sc as plsc

_NC = 2
_NS = 16
_NW = _NC * _NS
_L = 16


def _sqrt_pos(t):
    bits = lax.bitcast_convert_type(t, jnp.int32)
    y = lax.bitcast_convert_type(
        jnp.int32(0x5F3759DF) - lax.shift_right_logical(bits, 1), jnp.float32)
    for _ in range(3):
        y = y * (1.5 - 0.5 * t * y * y)
    return jnp.where(t > 0, t * y, 0.0)


def _log(y):
    bits = lax.bitcast_convert_type(y, jnp.int32)
    e = lax.shift_right_logical(bits, 23) - 127
    m = lax.bitcast_convert_type(
        jnp.bitwise_or(jnp.bitwise_and(bits, 0x007FFFFF), 0x3F800000), jnp.float32)
    big = m > 1.4142135
    m = jnp.where(big, 0.5 * m, m)
    ef = (e + jnp.where(big, 1, 0)).astype(jnp.float32)
    z = (m - 1.0) / (m + 1.0)
    z2 = z * z
    p = z * (2.0 + z2 * (2.0 / 3.0 + z2 * (2.0 / 5.0 + z2 * (2.0 / 7.0 + z2 * (2.0 / 9.0)))))
    return ef * 0.69314718 + p


def _sc_partials(i0, i1, values, wflat, d_stride, d_dim):
    B = i0.shape[0]
    D = d_dim
    bpw = B // _NW
    G = bpw // _L
    mesh = plsc.VectorSubcoreMesh(core_axis_name="c", subcore_axis_name="s")

    @functools.partial(
        pl.kernel,
        out_type=jax.ShapeDtypeStruct((_NW, _L), jnp.float32),
        mesh=mesh,
        compiler_params=pltpu.CompilerParams(
            needs_layout_passes=False, use_tc_tiling_on_sc=False),
        scratch_types=[
            pltpu.VMEM((bpw,), jnp.int32),
            pltpu.VMEM((bpw,), jnp.int32),
            pltpu.VMEM((bpw,), jnp.float32),
            pltpu.VMEM((bpw * D,), jnp.int32),
            pltpu.VMEM((bpw * D,), jnp.int32),
            pltpu.VMEM((bpw * D,), jnp.float32),
            pltpu.VMEM((bpw * D,), jnp.float32),
            pltpu.VMEM((_L,), jnp.float32),
            pltpu.SemaphoreType.DMA,
            pltpu.SemaphoreType.DMA,
        ],
    )
    def body(i0_hbm, i1_hbm, vals_hbm, w_hbm, out_hbm,
             i0_v, i1_v, vals_v, li_u, li_v, gu, gv, acc_v, s0, s1):
        wid = lax.axis_index("s") * _NC + lax.axis_index("c")
        base = wid * bpw
        pltpu.sync_copy(i0_hbm.at[pl.ds(base, bpw)], i0_v)
        pltpu.sync_copy(i1_hbm.at[pl.ds(base, bpw)], i1_v)
        pltpu.sync_copy(vals_hbm.at[pl.ds(base, bpw)], vals_v)

        def build(g, _):
            r0 = i0_v[pl.ds(g * _L, _L)]
            r1 = i1_v[pl.ds(g * _L, _L)]
            for d in range(D):
                off = jnp.int32(d * d_stride)
                li_u[pl.ds(d * bpw + g * _L, _L)] = r0 + off
                li_v[pl.ds(d * bpw + g * _L, _L)] = r1 + off
            return 0

        lax.fori_loop(0, G, build, 0)

        cp0 = pltpu.async_copy(w_hbm.at[li_u], gu, s0)
        cp1 = pltpu.async_copy(w_hbm.at[li_v], gv, s1)
        cp0.wait()
        cp1.wait()

        zero = jnp.zeros((_L,), jnp.float32)

        def g_body(g, acc):
            su = zero
            sv = zero
            sd = zero
            for d in range(D):
                u = gu[pl.ds(d * bpw + g * _L, _L)]
                v = gv[pl.ds(d * bpw + g * _L, _L)]
                su = su + u * u
                sv = sv + v * v
                du = u - v
                sd = sd + du * du
            vals = vals_v[pl.ds(g * _L, _L)]
            x = 1.0 + (2.0 * sd) / ((1.0 - su) * (1.0 - sv))
            dist = _log(x + _sqrt_pos(x * x - 1.0))
            q = dist / vals - 1.0
            return acc + jnp.exp(2.0 * (1.0 - vals)) * q * q

        acc_v[...] = lax.fori_loop(0, G, g_body, zero)
        pltpu.sync_copy(acc_v, out_hbm.at[wid])

    return body(i0, i1, values, wflat)


def _tc_rowmajor(wt, n_rows, d_dim, n_chunk):
    n_blocks = -(-n_rows // n_chunk)
    n_pad = n_blocks * n_chunk

    def t_body(in_ref, o_ref):
        o_ref[...] = in_ref[...].reshape(8, n_chunk // 128, 128)

    out = pl.pallas_call(
        t_body,
        grid=(n_blocks, d_dim // 8),
        in_specs=[pl.BlockSpec((8, n_chunk), lambda i, j: (j, i))],
        out_specs=pl.BlockSpec((8, n_chunk // 128, 128), lambda i, j: (j, i, 0)),
        out_shape=jax.ShapeDtypeStruct((d_dim, n_pad // 128, 128), jnp.float32),
    )(wt)
    return out.reshape(d_dim * n_pad), n_pad


def _tc_sum(partials, inv_pairs):
    def sum_body(x_ref, o_ref):
        o_ref[0, 0] = jnp.sum(x_ref[...]) * inv_pairs

    out = pl.pallas_call(
        sum_body,
        out_shape=jax.ShapeDtypeStruct((1, 1), jnp.float32),
        out_specs=pl.BlockSpec(memory_space=pltpu.SMEM),
    )(partials)
    return out[0, 0]


def kernel(idx, values, w, scale):
    del scale
    N, D = w.shape
    i0 = idx[:, 0].astype(jnp.int32)
    i1 = idx[:, 1].astype(jnp.int32)
    wt = jnp.swapaxes(w, 0, 1)
    wflat, n_pad = _tc_rowmajor(wt, N, D, 8192)
    partials = _sc_partials(i0, i1, values, wflat, n_pad, D)
    inv_pairs = 2.0 / (float(N) * float(N - 1))
    return _tc_sum(partials, inv_pairs)


# speedup vs baseline: 2.5138x; 2.5138x over previous
import functools

import jax
import jax.numpy as jnp
from jax import lax
from jax.experimental import pallas as pl
from jax.experimental.pallas import tpu as pltpu
from jax.experimental.pallas import tpu_sc as plsc

_NC = 2
_NS = 16
_NW = _NC * _NS
_L = 16


def _sqrt_pos(t):
    bits = lax.bitcast_convert_type(t, jnp.int32)
    y = lax.bitcast_convert_type(
        jnp.int32(0x5F3759DF) - lax.shift_right_logical(bits, 1), jnp.float32)
    for _ in range(3):
        y = y * (1.5 - 0.5 * t * y * y)
    return jnp.where(t > 0, t * y, 0.0)


def _log(y):
    bits = lax.bitcast_convert_type(y, jnp.int32)
    e = lax.shift_right_logical(bits, 23) - 127
    m = lax.bitcast_convert_type(
        jnp.bitwise_or(jnp.bitwise_and(bits, 0x007FFFFF), 0x3F800000), jnp.float32)
    big = m > 1.4142135
    m = jnp.where(big, 0.5 * m, m)
    ef = (e + jnp.where(big, 1, 0)).astype(jnp.float32)
    z = (m - 1.0) / (m + 1.0)
    z2 = z * z
    p = z * (2.0 + z2 * (2.0 / 3.0 + z2 * (2.0 / 5.0 + z2 * (2.0 / 7.0 + z2 * (2.0 / 9.0)))))
    return ef * 0.69314718 + p


def _sc_partials(i0, i1, values, wflat, d_stride, d_dim):
    B = i0.shape[0]
    D = d_dim
    bpw = B // _NW
    G = bpw // _L
    mesh = plsc.VectorSubcoreMesh(core_axis_name="c", subcore_axis_name="s")

    @functools.partial(
        pl.kernel,
        out_type=jax.ShapeDtypeStruct((_NW, _L), jnp.float32),
        mesh=mesh,
        compiler_params=pltpu.CompilerParams(
            needs_layout_passes=False, use_tc_tiling_on_sc=False),
        scratch_types=[
            pltpu.VMEM((bpw,), jnp.int32),
            pltpu.VMEM((bpw,), jnp.int32),
            pltpu.VMEM((bpw,), jnp.float32),
            pltpu.VMEM((bpw * D,), jnp.int32),
            pltpu.VMEM((bpw * D,), jnp.int32),
            pltpu.VMEM((bpw * D,), jnp.float32),
            pltpu.VMEM((bpw * D,), jnp.float32),
            pltpu.VMEM((_L,), jnp.float32),
            pltpu.SemaphoreType.DMA,
            pltpu.SemaphoreType.DMA,
        ],
    )
    def body(i0_hbm, i1_hbm, vals_hbm, w_hbm, out_hbm,
             i0_v, i1_v, vals_v, li_u, li_v, gu, gv, acc_v, s0, s1):
        wid = lax.axis_index("s") * _NC + lax.axis_index("c")
        base = wid * bpw
        pltpu.sync_copy(i0_hbm.at[pl.ds(base, bpw)], i0_v)
        pltpu.sync_copy(i1_hbm.at[pl.ds(base, bpw)], i1_v)
        pltpu.sync_copy(vals_hbm.at[pl.ds(base, bpw)], vals_v)

        def build(g, _):
            r0 = i0_v[pl.ds(g * _L, _L)]
            r1 = i1_v[pl.ds(g * _L, _L)]
            for d in range(D):
                off = jnp.int32(d * d_stride)
                li_u[pl.ds(d * bpw + g * _L, _L)] = r0 + off
                li_v[pl.ds(d * bpw + g * _L, _L)] = r1 + off
            return 0

        lax.fori_loop(0, G, build, 0)

        cp0 = pltpu.async_copy(w_hbm.at[li_u], gu, s0)
        cp1 = pltpu.async_copy(w_hbm.at[li_v], gv, s1)
        cp0.wait()
        cp1.wait()

        zero = jnp.zeros((_L,), jnp.float32)

        def g_body(g, acc):
            su = zero
            sv = zero
            sd = zero
            for d in range(D):
                u = gu[pl.ds(d * bpw + g * _L, _L)]
                v = gv[pl.ds(d * bpw + g * _L, _L)]
                su = su + u * u
                sv = sv + v * v
                du = u - v
                sd = sd + du * du
            vals = vals_v[pl.ds(g * _L, _L)]
            x = 1.0 + (2.0 * sd) / ((1.0 - su) * (1.0 - sv))
            dist = _log(x + _sqrt_pos(x * x - 1.0))
            q = dist / vals - 1.0
            return acc + jnp.exp(2.0 * (1.0 - vals)) * q * q

        acc_v[...] = lax.fori_loop(0, G, g_body, zero)
        pltpu.sync_copy(acc_v, out_hbm.at[wid])

    return body(i0, i1, values, wflat)


def _tc_rowmajor(wt, n_rows, d_dim, n_chunk):
    n_blocks = -(-n_rows // n_chunk)
    n_pad = n_blocks * n_chunk

    def t_body(in_ref, o_ref):
        o_ref[...] = in_ref[...].reshape(d_dim, n_chunk // 128, 128)

    out = pl.pallas_call(
        t_body,
        grid=(n_blocks,),
        in_specs=[pl.BlockSpec((d_dim, n_chunk), lambda i: (0, i))],
        out_specs=pl.BlockSpec((d_dim, n_chunk // 128, 128), lambda i: (0, i, 0)),
        out_shape=jax.ShapeDtypeStruct((d_dim, n_pad // 128, 128), jnp.float32),
    )(wt)
    return out.reshape(d_dim * n_pad), n_pad


def _tc_sum(partials, inv_pairs):
    def sum_body(x_ref, o_ref):
        o_ref[0, 0] = jnp.sum(x_ref[...]) * inv_pairs

    out = pl.pallas_call(
        sum_body,
        out_shape=jax.ShapeDtypeStruct((1, 1), jnp.float32),
        out_specs=pl.BlockSpec(memory_space=pltpu.SMEM),
    )(partials)
    return out[0, 0]


def kernel(idx, values, w, scale):
    del scale
    N, D = w.shape
    i0 = idx[:, 0].astype(jnp.int32)
    i1 = idx[:, 1].astype(jnp.int32)
    wt = jnp.swapaxes(w, 0, 1)
    wflat, n_pad = _tc_rowmajor(wt, N, D, 32768)
    partials = _sc_partials(i0, i1, values, wflat, n_pad, D)
    inv_pairs = 2.0 / (float(N) * float(N - 1))
    return _tc_sum(partials, inv_pairs)
